# Initial kernel scaffold; baseline (speedup 1.0000x reference)
import functools

import jax
import jax.numpy as jnp
from jax import lax
from jax.experimental import pallas as pl
from jax.experimental.pallas import tpu as pltpu
from jax.experimental.pallas import tpu_sc as plsc

N = 10000
D = 128
E = 320000
NW = 32
K = 128
CH = 80
EPAD = NW * CH * K
NPAD = 10240
RPT = NPAD // NW

_MESH = plsc.VectorSubcoreMesh(core_axis_name="c", subcore_axis_name="s")


def _hist_body(dst_hbm, out_hbm, hist_sh, slab_v, ones_v, zrow_v):
    c = lax.axis_index("c")
    s = lax.axis_index("s")
    w = s * 2 + c

    def fill_ones(i, _):
        ones_v[pl.ds(i * 16, 16)] = jnp.full((16,), 1.0, jnp.float32)
        return 0

    lax.fori_loop(0, 8, fill_ones, 0)

    def fill_zero(i, _):
        zrow_v[pl.ds(i * 16, 16)] = jnp.zeros((16,), jnp.float32)
        return 0

    lax.fori_loop(0, 40, fill_zero, 0)

    pltpu.sync_copy(zrow_v, hist_sh.at[pl.ds(s * 640, 640)])
    plsc.subcore_barrier()

    pltpu.sync_copy(dst_hbm.at[w], slab_v)

    def scat(j, _):
        pltpu.sync_copy(ones_v, hist_sh.at[slab_v.at[j]], add=True)
        return 0

    lax.fori_loop(0, CH, scat, 0)
    plsc.subcore_barrier()
    pltpu.sync_copy(hist_sh.at[pl.ds(s * 640, 640)],
                    out_hbm.at[c, pl.ds(s * 640, 640)])


_hist_call = pl.kernel(
    _hist_body,
    out_type=jax.ShapeDtypeStruct((2, NPAD), jnp.float32),
    mesh=_MESH,
    scratch_types=[
        pltpu.VMEM_SHARED((NPAD,), jnp.float32),
        pltpu.VMEM((CH, K), jnp.int32),
        pltpu.VMEM((K,), jnp.float32),
        pltpu.VMEM((640,), jnp.float32),
    ],
)


def _scale_body(x_ref, w_ref, h0_ref, h1_ref, g_ref, dis_ref):
    deg = h0_ref[...] + h1_ref[...] + 1.0
    dis = lax.rsqrt(deg)
    h = jnp.dot(x_ref[...], w_ref[...], preferred_element_type=jnp.float32)
    g_ref[...] = h * dis
    dis_ref[...] = dis


_scale_call = pl.pallas_call(
    _scale_body,
    out_shape=(
        jax.ShapeDtypeStruct((N, D), jnp.float32),
        jax.ShapeDtypeStruct((N, 1), jnp.float32),
    ),
)


GRP = 8
NG = CH // GRP


def _agg_body(g_hbm, src_hbm, dst_hbm, out_hbm,
              acc_sh, srcs_v, dsts_v, rows0_v, rows1_v, zbuf_v, sem0, sem1):
    c = lax.axis_index("c")
    s = lax.axis_index("s")
    w = s * 2 + c

    def zrow(i, _):
        for c8 in range(8):
            zbuf_v[i, pl.ds(c8 * 16, 16)] = jnp.zeros((16,), jnp.float32)
        return 0

    lax.fori_loop(0, 64, zrow, 0)

    base = s * 640

    def zq(q, _):
        pltpu.sync_copy(zbuf_v, acc_sh.at[pl.ds(base + q * 64, 64)])
        return 0

    lax.fori_loop(0, 10, zq, 0)

    pltpu.sync_copy(src_hbm.at[w], srcs_v)
    pltpu.sync_copy(dst_hbm.at[w], dsts_v)
    plsc.subcore_barrier()

    pltpu.async_copy(g_hbm.at[srcs_v.at[0]], rows0_v, sem0)

    def grp(gidx, _):
        for k in range(GRP):
            j = gidx * GRP + k
            cur, csem = (rows0_v, sem0) if k % 2 == 0 else (rows1_v, sem1)
            nxt, nsem = (rows1_v, sem1) if k % 2 == 0 else (rows0_v, sem0)
            jn = jnp.minimum(j + 1, CH - 1)
            pltpu.async_copy(g_hbm.at[srcs_v.at[jn]], nxt, nsem)
            pltpu.make_async_copy(g_hbm.at[srcs_v.at[j]], cur, csem).wait()
            pltpu.sync_copy(cur, acc_sh.at[dsts_v.at[j]], add=True)
        return 0

    lax.fori_loop(0, NG, grp, 0)
    pltpu.make_async_copy(g_hbm.at[srcs_v.at[0]], rows0_v, sem0).wait()
    plsc.subcore_barrier()

    def outq(q, _):
        b0 = base + q * 64
        pltpu.sync_copy(acc_sh.at[pl.ds(b0, 64)], out_hbm.at[c, pl.ds(b0, 64)])
        return 0

    lax.fori_loop(0, 10, outq, 0)


_agg_call = pl.kernel(
    _agg_body,
    out_type=jax.ShapeDtypeStruct((2, NPAD, D), jnp.float32),
    mesh=_MESH,
    scratch_types=[
        pltpu.VMEM_SHARED((NPAD, D), jnp.float32),
        pltpu.VMEM((CH, K), jnp.int32),
        pltpu.VMEM((CH, K), jnp.int32),
        pltpu.VMEM((K, D), jnp.float32),
        pltpu.VMEM((K, D), jnp.float32),
        pltpu.VMEM((64, D), jnp.float32),
        pltpu.SemaphoreType.DMA,
        pltpu.SemaphoreType.DMA,
    ],
)


def _final_body(acc_ref, g_ref, dis_ref, b_ref, out_ref):
    t = acc_ref[0, :N, :] + acc_ref[1, :N, :] + g_ref[...]
    out_ref[...] = jnp.maximum(t * dis_ref[...] + b_ref[...], 0.0)


_final_call = pl.pallas_call(
    _final_body,
    out_shape=jax.ShapeDtypeStruct((N, D), jnp.float32),
)


def kernel(x, edge_index, W, b):
    src = edge_index[0].astype(jnp.int32)
    dst = edge_index[1].astype(jnp.int32)
    pad = EPAD - E
    srcp = jnp.concatenate([src, jnp.zeros((pad,), jnp.int32)])
    dstp = jnp.concatenate([dst, jnp.full((pad,), N, jnp.int32)])
    src3 = srcp.reshape(NW, CH, K)
    dst3 = dstp.reshape(NW, CH, K)

    hist2 = _hist_call(dst3)
    h0c = hist2[0, :N].reshape(N, 1)
    h1c = hist2[1, :N].reshape(N, 1)
    g, dis = _scale_call(x, W, h0c, h1c)
    acc = _agg_call(g, src3, dst3)
    out = _final_call(acc, g, dis, b.reshape(1, D))
    return out


# speedup vs baseline: 16.9475x; 16.9475x over previous
import functools

import jax
import jax.numpy as jnp
from jax import lax
from jax.experimental import pallas as pl
from jax.experimental.pallas import tpu as pltpu
from jax.experimental.pallas import tpu_sc as plsc

N = 10000
D = 128
E = 320000
NW = 32
K = 64
CH = 160
EPAD = NW * CH * K
NPAD = 10240
RPT = NPAD // NW

_MESH = plsc.VectorSubcoreMesh(core_axis_name="c", subcore_axis_name="s")


def _hist_body(dst_hbm, out_hbm, hist_sh, slab_v, ones_v, zrow_v):
    c = lax.axis_index("c")
    s = lax.axis_index("s")
    w = s * 2 + c

    def fill_ones(i, _):
        ones_v[pl.ds(i * 16, 16)] = jnp.full((16,), 1.0, jnp.float32)
        return 0

    lax.fori_loop(0, K // 16, fill_ones, 0)

    def fill_zero(i, _):
        zrow_v[pl.ds(i * 16, 16)] = jnp.zeros((16,), jnp.float32)
        return 0

    lax.fori_loop(0, 40, fill_zero, 0)

    pltpu.sync_copy(zrow_v, hist_sh.at[pl.ds(s * 640, 640)])
    plsc.subcore_barrier()

    pltpu.sync_copy(dst_hbm.at[w], slab_v)

    def scat(j, _):
        pltpu.sync_copy(ones_v, hist_sh.at[slab_v.at[j]], add=True)
        return 0

    lax.fori_loop(0, CH, scat, 0)
    plsc.subcore_barrier()
    pltpu.sync_copy(hist_sh.at[pl.ds(s * 640, 640)],
                    out_hbm.at[c, pl.ds(s * 640, 640)])


_hist_call = pl.kernel(
    _hist_body,
    out_type=jax.ShapeDtypeStruct((2, NPAD), jnp.float32),
    mesh=_MESH,
    scratch_types=[
        pltpu.VMEM_SHARED((NPAD,), jnp.float32),
        pltpu.VMEM((CH, K), jnp.int32),
        pltpu.VMEM((K,), jnp.float32),
        pltpu.VMEM((640,), jnp.float32),
    ],
)


def _scale_body(x_ref, w_ref, h0_ref, h1_ref, g_ref, dis_ref):
    deg = h0_ref[...] + h1_ref[...] + 1.0
    dis = lax.rsqrt(deg)
    h = jnp.dot(x_ref[...], w_ref[...], preferred_element_type=jnp.float32)
    g_ref[...] = h * dis
    dis_ref[...] = dis


_scale_call = pl.pallas_call(
    _scale_body,
    out_shape=(
        jax.ShapeDtypeStruct((N, D), jnp.float32),
        jax.ShapeDtypeStruct((N, 1), jnp.float32),
    ),
)


GRP = 8
NG = CH // GRP


def _src_slice(srcs_v, jrow, col):
    return srcs_v.at[jrow, pl.ds(col, K)]


def _agg_body(g_hbm, src_hbm, dst_hbm, z_hbm, out_hbm,
              acc_sh, srcs_v, dsts_v, rows0_v, rows1_v, sem0, sem1):
    c = lax.axis_index("c")
    s = lax.axis_index("s")
    w = s * 2 + c
    base = s * 640

    pltpu.sync_copy(z_hbm, acc_sh.at[pl.ds(base, 640)])

    pltpu.sync_copy(src_hbm.at[w], srcs_v)
    pltpu.sync_copy(dst_hbm.at[w], dsts_v)
    plsc.subcore_barrier()

    pltpu.async_copy(g_hbm.at[_src_slice(srcs_v, 0, 0)], rows0_v, sem0)

    def grp(gidx, _):
        for k in range(GRP):
            j = gidx * GRP + k
            cur, csem = (rows0_v, sem0) if k % 2 == 0 else (rows1_v, sem1)
            nxt, nsem = (rows1_v, sem1) if k % 2 == 0 else (rows0_v, sem0)
            jn = jnp.minimum(j + 1, CH - 1)
            pltpu.async_copy(
                g_hbm.at[_src_slice(srcs_v, jn // 2, (jn % 2) * K)], nxt, nsem)
            pltpu.make_async_copy(
                g_hbm.at[_src_slice(srcs_v, j // 2, (j % 2) * K)], cur,
                csem).wait()
            pltpu.sync_copy(cur, acc_sh.at[dsts_v.at[j]], add=True)
        return 0

    lax.fori_loop(0, NG, grp, 0)
    pltpu.make_async_copy(g_hbm.at[_src_slice(srcs_v, 0, 0)], rows0_v,
                          sem0).wait()
    plsc.subcore_barrier()

    def outq(q, _):
        b0 = base + q * 64
        pltpu.sync_copy(acc_sh.at[pl.ds(b0, 64)], out_hbm.at[c, pl.ds(b0, 64)])
        return 0

    lax.fori_loop(0, 10, outq, 0)


_agg_call = pl.kernel(
    _agg_body,
    out_type=jax.ShapeDtypeStruct((2, NPAD, D), jnp.float32),
    mesh=_MESH,
    scratch_types=[
        pltpu.VMEM_SHARED((NPAD, D), jnp.float32),
        pltpu.VMEM((CH // 2, 2 * K), jnp.int32),
        pltpu.VMEM((CH, K), jnp.int32),
        pltpu.VMEM((K, D), jnp.float32),
        pltpu.VMEM((K, D), jnp.float32),
        pltpu.SemaphoreType.DMA,
        pltpu.SemaphoreType.DMA,
    ],
)


def _final_body(acc_ref, g_ref, dis_ref, b_ref, out_ref):
    t = acc_ref[0, :N, :] + acc_ref[1, :N, :] + g_ref[...]
    out_ref[...] = jnp.maximum(t * dis_ref[...] + b_ref[...], 0.0)


_final_call = pl.pallas_call(
    _final_body,
    out_shape=jax.ShapeDtypeStruct((N, D), jnp.float32),
)


def kernel(x, edge_index, W, b):
    src = edge_index[0].astype(jnp.int32)
    dst = edge_index[1].astype(jnp.int32)
    pad = EPAD - E
    srcp = jnp.concatenate([src, jnp.zeros((pad,), jnp.int32)])
    dstp = jnp.concatenate([dst, jnp.full((pad,), N, jnp.int32)])
    src3 = srcp.reshape(NW, CH // 2, 2 * K)
    dst3 = dstp.reshape(NW, CH, K)
    zrows = jnp.zeros((NPAD // 16, D), jnp.float32)

    hist2 = _hist_call(dst3)
    h0c = hist2[0, :N].reshape(N, 1)
    h1c = hist2[1, :N].reshape(N, 1)
    g, dis = _scale_call(x, W, h0c, h1c)
    acc = _agg_call(g, src3, dst3, zrows)
    out = _final_call(acc, g, dis, b.reshape(1, D))
    return out


# speedup vs baseline: 17.2208x; 1.0161x over previous
import functools

import jax
import jax.numpy as jnp
from jax import lax
from jax.experimental import pallas as pl
from jax.experimental.pallas import tpu as pltpu
from jax.experimental.pallas import tpu_sc as plsc

N = 10000
D = 128
E = 320000
NW = 32
K = 64
CH = 160
EPAD = NW * CH * K
NPAD = 10240
RPT = NPAD // NW

_MESH = plsc.VectorSubcoreMesh(core_axis_name="c", subcore_axis_name="s")


def _hist_body(dst_hbm, out_hbm, hist_sh, slab_v, ones_v, zrow_v):
    c = lax.axis_index("c")
    s = lax.axis_index("s")
    w = s * 2 + c

    def fill_ones(i, _):
        ones_v[pl.ds(i * 16, 16)] = jnp.full((16,), 1.0, jnp.float32)
        return 0

    lax.fori_loop(0, K // 16, fill_ones, 0)

    def fill_zero(i, _):
        zrow_v[pl.ds(i * 16, 16)] = jnp.zeros((16,), jnp.float32)
        return 0

    lax.fori_loop(0, 40, fill_zero, 0)

    pltpu.sync_copy(zrow_v, hist_sh.at[pl.ds(s * 640, 640)])
    plsc.subcore_barrier()

    pltpu.sync_copy(dst_hbm.at[w], slab_v)

    def scat(j, _):
        pltpu.sync_copy(ones_v, hist_sh.at[slab_v.at[j]], add=True)
        return 0

    lax.fori_loop(0, CH, scat, 0)
    plsc.subcore_barrier()
    pltpu.sync_copy(hist_sh.at[pl.ds(s * 640, 640)],
                    out_hbm.at[c, pl.ds(s * 640, 640)])


_hist_call = pl.kernel(
    _hist_body,
    out_type=jax.ShapeDtypeStruct((2, NPAD), jnp.float32),
    mesh=_MESH,
    scratch_types=[
        pltpu.VMEM_SHARED((NPAD,), jnp.float32),
        pltpu.VMEM((CH, K), jnp.int32),
        pltpu.VMEM((K,), jnp.float32),
        pltpu.VMEM((640,), jnp.float32),
    ],
)


def _scale_body(x_ref, w_ref, h0_ref, h1_ref, g_ref, dis_ref):
    deg = h0_ref[...] + h1_ref[...] + 1.0
    dis = lax.rsqrt(deg)
    h = jnp.dot(x_ref[...], w_ref[...], preferred_element_type=jnp.float32)
    g_ref[...] = h * dis
    dis_ref[...] = dis


_scale_call = pl.pallas_call(
    _scale_body,
    out_shape=(
        jax.ShapeDtypeStruct((N, D), jnp.float32),
        jax.ShapeDtypeStruct((N, 1), jnp.float32),
    ),
)


NSLOT = 4
PIECE = 8
NSSG = CH // 16


def _agg_body(g_hbm, src_hbm, dst_hbm, z_hbm, out_hbm,
              acc_sh, srcs_v, p0_v, p1_v, rows_v,
              gsems, ssems, isems):
    c = lax.axis_index("c")
    s = lax.axis_index("s")
    w = s * 2 + c
    base = s * 640

    def slot(j_static_mod):
        return rows_v.at[pl.ds(j_static_mod * K, K)]

    def start_gather(j, sl):
        pltpu.async_copy(
            g_hbm.at[srcs_v.at[j // 2, pl.ds((j % 2) * K, K)]],
            slot(sl), gsems[sl])

    def wait_gather(j, sl):
        pltpu.make_async_copy(
            g_hbm.at[srcs_v.at[j // 2, pl.ds((j % 2) * K, K)]],
            slot(sl), gsems[sl]).wait()

    def start_scatter(piece_v, r, sl):
        pltpu.async_copy(slot(sl), acc_sh.at[piece_v.at[r]], ssems[sl],
                         add=True)

    def wait_scatter(piece_v, r, sl):
        pltpu.make_async_copy(slot(sl), acc_sh.at[piece_v.at[r]],
                              ssems[sl]).wait()

    def start_piece(p, pv, b):
        pltpu.async_copy(dst_hbm.at[w, p], pv, isems[b])

    def wait_piece(p, pv, b):
        pltpu.make_async_copy(dst_hbm.at[w, p], pv, isems[b]).wait()

    pltpu.sync_copy(z_hbm, acc_sh.at[pl.ds(base, 640)])
    pltpu.sync_copy(src_hbm.at[w], srcs_v)
    plsc.subcore_barrier()

    start_piece(0, p0_v, 0)
    start_gather(0, 0)
    start_gather(1, 1)

    def ssg_body(ssg, _):
        j0 = ssg * 16
        for k in range(16):
            j = j0 + k
            sl = k % 4
            pv, r = (p0_v, k) if k < 8 else (p1_v, k - 8)
            b = 0 if k < 8 else 1
            if k == 0:
                wait_piece(2 * ssg, p0_v, 0)
            if k == 2:
                start_piece(2 * ssg + 1, p1_v, 1)
            if k == 8:
                wait_piece(2 * ssg + 1, p1_v, 1)
            if k == 10:
                start_piece(jnp.minimum(2 * ssg + 2, 2 * NSSG - 1), p0_v, 0)
            sl2 = (k + 2) % 4
            if k < 2:
                @pl.when(ssg > 0)
                def _():
                    pk = (16 + k + 2 - 4) % 16
                    pvp = p0_v if pk < 8 else p1_v
                    wait_scatter(pvp, pk % 8, sl2)
            else:
                pk = k - 2
                pvp, rp = (p0_v, pk) if pk < 8 else (p1_v, pk - 8)
                wait_scatter(pvp, rp, sl2)
            start_gather(jnp.minimum(j + 2, CH - 1), sl2)
            wait_gather(j, sl)
            start_scatter(pv, r, sl)
        return 0

    lax.fori_loop(0, NSSG, ssg_body, 0)

    wait_scatter(p1_v, 6, (CH - 2) % 4)
    wait_scatter(p1_v, 7, (CH - 1) % 4)
    wait_gather(CH - 1, CH % 4)
    wait_gather(CH - 1, (CH + 1) % 4)
    wait_piece(2 * NSSG - 1, p0_v, 0)
    plsc.subcore_barrier()

    def outq(q, _):
        b0 = base + q * 64
        pltpu.sync_copy(acc_sh.at[pl.ds(b0, 64)], out_hbm.at[c, pl.ds(b0, 64)])
        return 0

    lax.fori_loop(0, 10, outq, 0)


_agg_call = pl.kernel(
    _agg_body,
    out_type=jax.ShapeDtypeStruct((2, NPAD, D), jnp.float32),
    mesh=_MESH,
    scratch_types=[
        pltpu.VMEM_SHARED((NPAD, D), jnp.float32),
        pltpu.VMEM((CH // 2, 2 * K), jnp.int32),
        pltpu.VMEM((PIECE, K), jnp.int32),
        pltpu.VMEM((PIECE, K), jnp.int32),
        pltpu.VMEM((NSLOT * K, D), jnp.float32),
        [pltpu.SemaphoreType.DMA] * NSLOT,
        [pltpu.SemaphoreType.DMA] * NSLOT,
        [pltpu.SemaphoreType.DMA] * 2,
    ],
)


def _final_body(acc_ref, g_ref, dis_ref, b_ref, out_ref):
    t = acc_ref[0, :N, :] + acc_ref[1, :N, :] + g_ref[...]
    out_ref[...] = jnp.maximum(t * dis_ref[...] + b_ref[...], 0.0)


_final_call = pl.pallas_call(
    _final_body,
    out_shape=jax.ShapeDtypeStruct((N, D), jnp.float32),
)


def kernel(x, edge_index, W, b):
    src = edge_index[0].astype(jnp.int32)
    dst = edge_index[1].astype(jnp.int32)
    pad = EPAD - E
    srcp = jnp.concatenate([src, jnp.zeros((pad,), jnp.int32)])
    dstp = jnp.concatenate([dst, jnp.full((pad,), N, jnp.int32)])
    src3 = srcp.reshape(NW, CH // 2, 2 * K)
    dst3 = dstp.reshape(NW, CH, K)
    dst4 = dstp.reshape(NW, 2 * NSSG, PIECE, K)
    zrows = jnp.zeros((NPAD // 16, D), jnp.float32)

    hist2 = _hist_call(dst3)
    h0c = hist2[0, :N].reshape(N, 1)
    h1c = hist2[1, :N].reshape(N, 1)
    g, dis = _scale_call(x, W, h0c, h1c)
    acc = _agg_call(g, src3, dst4, zrows)
    out = _final_call(acc, g, dis, b.reshape(1, D))
    return out
